# baseline (speedup 1.0000x reference)
import functools

import jax
import jax.numpy as jnp
from jax import lax
from jax.experimental import pallas as pl
from jax.experimental.pallas import tpu as pltpu
from jax.experimental.pallas import tpu_sc as plsc

_NC, _NS = 2, 16
_NW = _NC * _NS

_ROWS_PER_BLOCK = 128
_COL_BLOCK = 1024
_GATHER_CHUNK = 64
_COMBINE_TOKENS = 8


def _sc_mesh():
    return plsc.VectorSubcoreMesh(
        core_axis_name="c", subcore_axis_name="s",
        num_cores=_NC, num_subcores=_NS)


def _make_gather(P, W):
    rows_per_w = P // _NW
    chunk = _GATHER_CHUNK
    n_chunks = rows_per_w // chunk

    @functools.partial(
        pl.kernel, mesh=_sc_mesh(),
        out_type=jax.ShapeDtypeStruct((P, W), jnp.int32),
        scratch_types=[
            pltpu.VMEM((rows_per_w,), jnp.int32),
            pltpu.VMEM((chunk, W), jnp.int32),
            pltpu.VMEM((chunk, W), jnp.int32),
            pltpu.SemaphoreType.DMA,
            pltpu.SemaphoreType.DMA,
        ],
    )
    def k(x_hbm, tok_hbm, out_hbm, idx_v, buf0, buf1, sem0, sem1):
        wid = lax.axis_index("s") * _NC + lax.axis_index("c")
        base = wid * rows_per_w
        pltpu.sync_copy(tok_hbm.at[pl.ds(base, rows_per_w)], idx_v)
        bufs = (buf0, buf1)
        sems = (sem0, sem1)
        cps = [None, None]
        cps[0] = pltpu.async_copy(
            x_hbm.at[idx_v.at[pl.ds(0, chunk)]], bufs[0], sems[0])
        for ch in range(n_chunks):
            cur = ch % 2
            nxt = (ch + 1) % 2
            if ch + 1 < n_chunks:
                cps[nxt] = pltpu.async_copy(
                    x_hbm.at[idx_v.at[pl.ds((ch + 1) * chunk, chunk)]],
                    bufs[nxt], sems[nxt])
            cps[cur].wait()
            pltpu.sync_copy(bufs[cur], out_hbm.at[pl.ds(base + ch * chunk, chunk)])

    return k


def _make_combine(T, K, W):
    tok_per_w = T // _NW
    ct = _COMBINE_TOKENS
    n_chunks = tok_per_w // ct
    rows_per_chunk = ct * K
    bf = jnp.bfloat16

    @functools.partial(
        pl.kernel, mesh=_sc_mesh(),
        out_type=jax.ShapeDtypeStruct((T, W), jnp.int32),
        scratch_types=[
            pltpu.VMEM((tok_per_w * K,), jnp.int32),
            pltpu.VMEM((tok_per_w * K,), jnp.float32),
            pltpu.VMEM((rows_per_chunk, W), jnp.int32),
            pltpu.VMEM((ct, W), jnp.int32),
            pltpu.SemaphoreType.DMA,
        ],
        compiler_params=pltpu.CompilerParams(needs_layout_passes=False),
    )
    def k(y_hbm, inv_hbm, g_hbm, out_hbm, inv_v, g_v, rows_v, out_v, sem):
        wid = lax.axis_index("s") * _NC + lax.axis_index("c")
        tbase = wid * tok_per_w
        pbase = tbase * K
        pltpu.sync_copy(inv_hbm.at[pl.ds(pbase, tok_per_w * K)], inv_v)
        pltpu.sync_copy(g_hbm.at[pl.ds(pbase, tok_per_w * K)], g_v)
        iota16 = lax.iota(jnp.int32, 16)
        for ch in range(n_chunks):
            pltpu.async_copy(
                y_hbm.at[inv_v.at[pl.ds(ch * rows_per_chunk, rows_per_chunk)]],
                rows_v, sem).wait()
            gchunk = g_v[pl.ds(ch * rows_per_chunk, 16)]
            for t in range(ct):
                g0s = jnp.sum(jnp.where(iota16 == K * t, gchunk, 0.0))
                g1s = jnp.sum(jnp.where(iota16 == K * t + 1, gchunk, 0.0))
                g0f = jnp.broadcast_to(g0s, (16,))
                g1f = jnp.broadcast_to(g1s, (16,))
                g0 = plsc.pack(g0f, g0f, format=plsc.PackFormat.INTERLEAVED)
                g1 = plsc.pack(g1f, g1f, format=plsc.PackFormat.INTERLEAVED)

                def body(kk, _, t=t, g0=g0, g1=g1):
                    off = pl.multiple_of(kk * 16, 16)
                    a = plsc.bitcast(rows_v[K * t, pl.ds(off, 16)], bf)
                    b = plsc.bitcast(rows_v[K * t + 1, pl.ds(off, 16)], bf)
                    out_v[t, pl.ds(off, 16)] = plsc.bitcast(g0 * a + g1 * b,
                                                            jnp.int32)
                    return 0

                lax.fori_loop(0, W // 16, body, 0)
            pltpu.sync_copy(out_v, out_hbm.at[pl.ds(tbase + ch * ct, ct)])

    return k


def _gemm_body(sched_ref, x_ref, w_ref, o_ref):
    j = pl.program_id(1)
    blk = sched_ref[0, j]
    lo = sched_ref[1, j]
    hi = sched_ref[2, j]
    first = sched_ref[4, j]
    rows = blk * _ROWS_PER_BLOCK + lax.broadcasted_iota(
        jnp.int32, (_ROWS_PER_BLOCK, 1), 0)
    mask = (rows >= lo) & (rows < hi)
    xm = jnp.where(mask, x_ref[...], jnp.zeros_like(x_ref))
    y = jnp.dot(xm, w_ref[0], preferred_element_type=jnp.float32)
    y = y.astype(o_ref.dtype)

    @pl.when(first == 1)
    def _init():
        o_ref[...] = y

    @pl.when(first == 0)
    def _acc():
        o_ref[...] = o_ref[...] + y


def _build_schedule(expert_offsets, P, E, n_items_max):
    offsets = expert_offsets.astype(jnp.int32)
    nb = P // _ROWS_PER_BLOCK
    b = jnp.arange(nb, dtype=jnp.int32)
    r0 = b * _ROWS_PER_BLOCK
    r1 = r0 + (_ROWS_PER_BLOCK - 1)
    e_start = jnp.searchsorted(offsets, r0, side="right").astype(jnp.int32)
    e_end = jnp.searchsorted(offsets, r1, side="right").astype(jnp.int32)
    ipb = e_end - e_start + 1
    cum = jnp.cumsum(ipb)
    n_items = cum[-1]
    j = jnp.arange(n_items_max, dtype=jnp.int32)
    valid = j < n_items
    blk = jnp.minimum(
        jnp.searchsorted(cum, j, side="right").astype(jnp.int32), nb - 1)
    item_start = cum[blk] - ipb[blk]
    e_j = e_start[blk] + (j - item_start)
    e_c = jnp.clip(e_j, 0, E - 1)
    starts = jnp.concatenate(
        [jnp.zeros((1,), jnp.int32), offsets[:-1]])
    lo = jnp.where(valid, starts[e_c], 0)
    hi = jnp.where(valid, offsets[e_c], 0)
    ew = jnp.where(valid, e_c, e_end[nb - 1])
    first = ((j == item_start) & valid).astype(jnp.int32)
    blk_f = jnp.where(valid, blk, nb - 1)
    return jnp.stack([blk_f, lo, hi, ew, first])


def kernel(x, bin_ids, indices, padded_block_idxs, expert_offsets, gates, weight):
    T, D_IN = x.shape
    E, _, D_OUT = weight.shape
    _, K = gates.shape
    P = indices.shape[0]
    n_items_max = P // _ROWS_PER_BLOCK + (E - 1)
    n_col = D_OUT // _COL_BLOCK

    indices = indices.astype(jnp.int32)
    tok = indices // K
    inv = jnp.zeros((P,), jnp.int32).at[indices].set(
        jnp.arange(P, dtype=jnp.int32), unique_indices=True,
        mode="promise_in_bounds")
    gflat = gates.reshape(-1).astype(jnp.float32)
    sched = _build_schedule(expert_offsets, P, E, n_items_max)

    x_i32 = lax.bitcast_convert_type(
        x.reshape(T, D_IN // 2, 2), jnp.int32)
    xg = _make_gather(P, D_IN // 2)(x_i32, tok)
    xg = lax.bitcast_convert_type(xg, x.dtype).reshape(P, D_IN)

    grid_spec = pltpu.PrefetchScalarGridSpec(
        num_scalar_prefetch=1,
        grid=(n_col, n_items_max),
        in_specs=[
            pl.BlockSpec((_ROWS_PER_BLOCK, D_IN), lambda c, j, s: (s[0, j], 0)),
            pl.BlockSpec((1, D_IN, _COL_BLOCK), lambda c, j, s: (s[3, j], 0, c)),
        ],
        out_specs=pl.BlockSpec(
            (_ROWS_PER_BLOCK, _COL_BLOCK), lambda c, j, s: (s[0, j], c)),
    )
    y = pl.pallas_call(
        _gemm_body,
        grid_spec=grid_spec,
        out_shape=jax.ShapeDtypeStruct((P, D_OUT), x.dtype),
        compiler_params=pltpu.CompilerParams(
            dimension_semantics=("arbitrary", "arbitrary")),
    )(sched, xg, weight)

    y_i32 = lax.bitcast_convert_type(
        y.reshape(P, D_OUT // 2, 2), jnp.int32)
    out_i32 = _make_combine(T, K, D_OUT // 2)(y_i32, inv, gflat)
    out = lax.bitcast_convert_type(out_i32, x.dtype).reshape(T, D_OUT)
    return out


# speedup vs baseline: 2.7865x; 2.7865x over previous
import functools

import jax
import jax.numpy as jnp
from jax import lax
from jax.experimental import pallas as pl
from jax.experimental.pallas import tpu as pltpu
from jax.experimental.pallas import tpu_sc as plsc

_NC, _NS = 2, 16
_NW = _NC * _NS

_ROWS_PER_BLOCK = 128
_COL_BLOCK = 4096
_GATHER_CHUNK = 32
_COMBINE_TOKENS = 4


def _sc_mesh():
    return plsc.VectorSubcoreMesh(
        core_axis_name="c", subcore_axis_name="s",
        num_cores=_NC, num_subcores=_NS)


def _make_gather(P, D):
    rows_per_w = P // _NW
    chunk = _GATHER_CHUNK
    n_chunks = rows_per_w // chunk

    @functools.partial(
        pl.kernel, mesh=_sc_mesh(),
        out_type=jax.ShapeDtypeStruct((P, D), jnp.float32),
        scratch_types=[
            pltpu.VMEM((rows_per_w,), jnp.int32),
            pltpu.VMEM((chunk, D), jnp.float32),
            pltpu.VMEM((chunk, D), jnp.float32),
            pltpu.SemaphoreType.DMA,
            pltpu.SemaphoreType.DMA,
            pltpu.SemaphoreType.DMA,
            pltpu.SemaphoreType.DMA,
        ],
    )
    def k(x_hbm, tok_hbm, out_hbm, idx_v, buf0, buf1, rs0, rs1, ws0, ws1):
        wid = lax.axis_index("s") * _NC + lax.axis_index("c")
        base = wid * rows_per_w
        pltpu.sync_copy(tok_hbm.at[pl.ds(base, rows_per_w)], idx_v)
        bufs = (buf0, buf1)
        rsems = (rs0, rs1)
        wsems = (ws0, ws1)
        rcp = [None, None]
        wcp = [None, None]
        rcp[0] = pltpu.async_copy(
            x_hbm.at[idx_v.at[pl.ds(0, chunk)]], bufs[0], rsems[0])
        for ch in range(n_chunks):
            cur = ch % 2
            nxt = (ch + 1) % 2
            if ch + 1 < n_chunks:
                if wcp[nxt] is not None:
                    wcp[nxt].wait()
                    wcp[nxt] = None
                rcp[nxt] = pltpu.async_copy(
                    x_hbm.at[idx_v.at[pl.ds((ch + 1) * chunk, chunk)]],
                    bufs[nxt], rsems[nxt])
            rcp[cur].wait()
            wcp[cur] = pltpu.async_copy(
                bufs[cur], out_hbm.at[pl.ds(base + ch * chunk, chunk)],
                wsems[cur])
        for b in range(2):
            if wcp[b] is not None:
                wcp[b].wait()

    return k


def _make_combine(T, K, D):
    tok_per_w = T // _NW
    ct = _COMBINE_TOKENS
    n_chunks = tok_per_w // ct
    rpc = ct * K

    @functools.partial(
        pl.kernel, mesh=_sc_mesh(),
        out_type=jax.ShapeDtypeStruct((T, D), jnp.float32),
        scratch_types=[
            pltpu.VMEM((tok_per_w * K,), jnp.int32),
            pltpu.VMEM((tok_per_w * K,), jnp.float32),
            pltpu.VMEM((rpc, D), jnp.float32),
            pltpu.VMEM((rpc, D), jnp.float32),
            pltpu.VMEM((ct, D), jnp.float32),
            pltpu.VMEM((ct, D), jnp.float32),
            pltpu.SemaphoreType.DMA,
            pltpu.SemaphoreType.DMA,
            pltpu.SemaphoreType.DMA,
            pltpu.SemaphoreType.DMA,
        ],
        compiler_params=pltpu.CompilerParams(needs_layout_passes=False),
    )
    def k(y_hbm, inv_hbm, g_hbm, out_hbm, inv_v, g_v,
          rows0, rows1, out0, out1, rs0, rs1, ws0, ws1):
        wid = lax.axis_index("s") * _NC + lax.axis_index("c")
        tbase = wid * tok_per_w
        pbase = tbase * K
        pltpu.sync_copy(inv_hbm.at[pl.ds(pbase, tok_per_w * K)], inv_v)
        pltpu.sync_copy(g_hbm.at[pl.ds(pbase, tok_per_w * K)], g_v)
        rows = (rows0, rows1)
        outs = (out0, out1)
        rsems = (rs0, rs1)
        wsems = (ws0, ws1)
        rcp = [None, None]
        wcp = [None, None]
        iota16 = lax.iota(jnp.int32, 16)
        rcp[0] = pltpu.async_copy(
            y_hbm.at[inv_v.at[pl.ds(0, rpc)]], rows[0], rsems[0])
        for ch in range(n_chunks):
            cur = ch % 2
            nxt = (ch + 1) % 2
            if ch + 1 < n_chunks:
                rcp[nxt] = pltpu.async_copy(
                    y_hbm.at[inv_v.at[pl.ds((ch + 1) * rpc, rpc)]],
                    rows[nxt], rsems[nxt])
            rcp[cur].wait()
            if wcp[cur] is not None:
                wcp[cur].wait()
                wcp[cur] = None
            gbase = pl.multiple_of(ch * rpc, 8)
            gchunk = g_v[pl.ds(gbase, 16)]
            rv = rows[cur]
            ov = outs[cur]
            for t in range(ct):
                g0s = jnp.sum(jnp.where(iota16 == K * t, gchunk, 0.0))
                g1s = jnp.sum(jnp.where(iota16 == K * t + 1, gchunk, 0.0))
                g0f = jnp.broadcast_to(g0s, (16,))
                g1f = jnp.broadcast_to(g1s, (16,))

                def body(kk, _, t=t, g0f=g0f, g1f=g1f, rv=rv, ov=ov):
                    off = pl.multiple_of(kk * 16, 16)
                    a = rv[K * t, pl.ds(off, 16)]
                    b = rv[K * t + 1, pl.ds(off, 16)]
                    ov[t, pl.ds(off, 16)] = g0f * a + g1f * b
                    return 0

                lax.fori_loop(0, D // 16, body, 0)
            wcp[cur] = pltpu.async_copy(
                ov, out_hbm.at[pl.ds(tbase + ch * ct, ct)], wsems[cur])
        for b in range(2):
            if wcp[b] is not None:
                wcp[b].wait()

    return k


def _gemm_body(sched_ref, x_ref, w_ref, o_ref, acc_ref):
    j = pl.program_id(0)
    blk = sched_ref[0, j]
    lo = sched_ref[1, j]
    hi = sched_ref[2, j]
    first = sched_ref[4, j]
    last = sched_ref[5, j]
    r0 = blk * _ROWS_PER_BLOCK
    full = (lo <= r0) & (hi >= r0 + _ROWS_PER_BLOCK)

    @pl.when(full)
    def _fast():
        xb = x_ref[...].astype(jnp.bfloat16)
        y = jnp.dot(xb, w_ref[0], preferred_element_type=jnp.float32)

        @pl.when(first == 1)
        def _():
            acc_ref[...] = y

        @pl.when(first == 0)
        def _():
            acc_ref[...] = acc_ref[...] + y

    @pl.when(jnp.logical_not(full))
    def _masked():
        rows = r0 + lax.broadcasted_iota(jnp.int32, (_ROWS_PER_BLOCK, 1), 0)
        mask = (rows >= lo) & (rows < hi)
        xb = jnp.where(mask, x_ref[...], 0.0).astype(jnp.bfloat16)
        y = jnp.dot(xb, w_ref[0], preferred_element_type=jnp.float32)

        @pl.when(first == 1)
        def _():
            acc_ref[...] = y

        @pl.when(first == 0)
        def _():
            acc_ref[...] = acc_ref[...] + y

    @pl.when(last == 1)
    def _flush():
        o_ref[...] = acc_ref[...]


def _build_schedule(expert_offsets, P, E, n_items_max):
    offsets = expert_offsets.astype(jnp.int32)
    nb = P // _ROWS_PER_BLOCK
    b = jnp.arange(nb, dtype=jnp.int32)
    r0 = b * _ROWS_PER_BLOCK
    r1 = r0 + (_ROWS_PER_BLOCK - 1)
    e_start = jnp.searchsorted(offsets, r0, side="right").astype(jnp.int32)
    e_end = jnp.searchsorted(offsets, r1, side="right").astype(jnp.int32)
    ipb = e_end - e_start + 1
    cum = jnp.cumsum(ipb)
    n_items = cum[-1]
    j = jnp.arange(n_items_max, dtype=jnp.int32)
    valid = j < n_items
    blk = jnp.minimum(
        jnp.searchsorted(cum, j, side="right").astype(jnp.int32), nb - 1)
    item_start = cum[blk] - ipb[blk]
    e_j = e_start[blk] + (j - item_start)
    e_c = jnp.clip(e_j, 0, E - 1)
    starts = jnp.concatenate(
        [jnp.zeros((1,), jnp.int32), offsets[:-1]])
    lo = jnp.where(valid, starts[e_c], 0)
    hi = jnp.where(valid, offsets[e_c], 0)
    ew = jnp.where(valid, e_c, e_end[nb - 1])
    first = ((j == item_start) & valid).astype(jnp.int32)
    blk_f = jnp.where(valid, blk, nb - 1)
    last = jnp.concatenate(
        [(blk_f[1:] != blk_f[:-1]).astype(jnp.int32),
         jnp.ones((1,), jnp.int32)])
    return jnp.stack([blk_f, lo, hi, ew, first, last])


def kernel(x, bin_ids, indices, padded_block_idxs, expert_offsets, gates, weight):
    T, D_IN = x.shape
    E, _, D_OUT = weight.shape
    _, K = gates.shape
    P = indices.shape[0]
    n_items_max = P // _ROWS_PER_BLOCK + (E - 1)

    indices = indices.astype(jnp.int32)
    tok = indices // K
    inv = jnp.zeros((P,), jnp.int32).at[indices].set(
        jnp.arange(P, dtype=jnp.int32), unique_indices=True,
        mode="promise_in_bounds")
    gflat = gates.reshape(-1).astype(jnp.float32)
    sched = _build_schedule(expert_offsets, P, E, n_items_max)

    xg = _make_gather(P, D_IN)(x.astype(jnp.float32), tok)

    grid_spec = pltpu.PrefetchScalarGridSpec(
        num_scalar_prefetch=1,
        grid=(n_items_max,),
        in_specs=[
            pl.BlockSpec((_ROWS_PER_BLOCK, D_IN), lambda j, s: (s[0, j], 0)),
            pl.BlockSpec((1, D_IN, _COL_BLOCK), lambda j, s: (s[3, j], 0, 0)),
        ],
        out_specs=pl.BlockSpec(
            (_ROWS_PER_BLOCK, _COL_BLOCK), lambda j, s: (s[0, j], 0)),
        scratch_shapes=[pltpu.VMEM((_ROWS_PER_BLOCK, _COL_BLOCK), jnp.float32)],
    )
    y = pl.pallas_call(
        _gemm_body,
        grid_spec=grid_spec,
        out_shape=jax.ShapeDtypeStruct((P, D_OUT), jnp.float32),
        compiler_params=pltpu.CompilerParams(
            dimension_semantics=("arbitrary",)),
    )(sched, xg, weight)

    out = _make_combine(T, K, D_OUT)(y, inv, gflat)
    return out.astype(x.dtype)
